# baseline (speedup 1.0000x reference)
import jax
import jax.numpy as jnp
from jax.experimental import pallas as pl
from jax.experimental.pallas import tpu as pltpu

N = 10000
D = 128

BM = 400
HM = BM // 2
NSTEPS = N // BM


def _fused_body(
    x_ref,
    w1t_ref,
    b1_ref,
    w2t_ref,
    b2_ref,
    adj_ref,
    out_ref,
    buf_ref,
    s2_ref,
    stage_ref,
    in_sems,
    out_sems,
):
    def in_copy(step, slot, half):
        return pltpu.make_async_copy(
            adj_ref.at[pl.ds(step * BM + half * HM, HM), :],
            buf_ref.at[slot, pl.ds(half * HM, HM), :],
            in_sems.at[slot, half],
        )

    def start_in(step, slot):
        in_copy(step, slot, 0).start()
        in_copy(step, slot, 1).start()

    def wait_in(step, slot):
        in_copy(step, slot, 0).wait()
        in_copy(step, slot, 1).wait()

    def out_copy(step, slot):
        return pltpu.make_async_copy(
            stage_ref.at[slot],
            out_ref.at[pl.ds(step * BM, BM), :],
            out_sems.at[slot],
        )

    start_in(0, 0)
    start_in(1, 1)

    s2_ref[...] = (
        jnp.dot(x_ref[...], w2t_ref[...], preferred_element_type=jnp.float32)
        + b2_ref[...]
    )

    def step(i, carry):
        slot = jax.lax.rem(i, 2)
        wait_in(i, slot)
        s1 = (
            jnp.dot(
                x_ref[pl.ds(i * BM, BM), :],
                w1t_ref[...],
                preferred_element_type=jnp.float32,
            )
            + b1_ref[...]
        )
        res = (
            jnp.dot(buf_ref[slot], s2_ref[...], preferred_element_type=jnp.float32)
            + s1
        )

        @pl.when(i >= 2)
        def _():
            out_copy(i - 2, slot).wait()

        stage_ref[slot] = res
        out_copy(i, slot).start()

        @pl.when(i + 2 < NSTEPS)
        def _():
            start_in(i + 2, slot)

        return carry

    jax.lax.fori_loop(0, NSTEPS, step, 0)
    out_copy(NSTEPS - 2, jax.lax.rem(NSTEPS - 2, 2)).wait()
    out_copy(NSTEPS - 1, jax.lax.rem(NSTEPS - 1, 2)).wait()


def kernel(input, adj, W1, b1, W2, b2):
    out = pl.pallas_call(
        _fused_body,
        grid=(1,),
        in_specs=[
            pl.BlockSpec((N, D), lambda i: (0, 0)),
            pl.BlockSpec((D, D), lambda i: (0, 0)),
            pl.BlockSpec((1, D), lambda i: (0, 0)),
            pl.BlockSpec((D, D), lambda i: (0, 0)),
            pl.BlockSpec((1, D), lambda i: (0, 0)),
            pl.BlockSpec(memory_space=pltpu.MemorySpace.HBM),
        ],
        out_specs=pl.BlockSpec(memory_space=pltpu.MemorySpace.HBM),
        out_shape=jax.ShapeDtypeStruct((N, D), jnp.float32),
        scratch_shapes=[
            pltpu.VMEM((2, BM, N), jnp.float32),
            pltpu.VMEM((N, D), jnp.float32),
            pltpu.VMEM((2, BM, D), jnp.float32),
            pltpu.SemaphoreType.DMA((2, 2)),
            pltpu.SemaphoreType.DMA((2,)),
        ],
        compiler_params=pltpu.CompilerParams(
            dimension_semantics=("arbitrary",),
        ),
    )(input, W1.T, b1.reshape(1, D), W2.T, b2.reshape(1, D), adj)
    return out


# speedup vs baseline: 1.0083x; 1.0083x over previous
import jax
import jax.numpy as jnp
from jax.experimental import pallas as pl
from jax.experimental.pallas import tpu as pltpu

N = 10000
D = 128
BM = 400
CK = 1280


def _fused_body(
    x_ref, w1t_ref, b1_ref, w2t_ref, b2_ref, adj_a_ref, adj_b_ref, out_ref, s2_ref
):
    i = pl.program_id(0)

    @pl.when(i == 0)
    def _():
        s2_ref[...] = (
            jnp.dot(x_ref[...], w2t_ref[...], preferred_element_type=jnp.float32)
            + b2_ref[...]
        )

    s1 = (
        jnp.dot(
            x_ref[pl.ds(i * BM, BM), :],
            w1t_ref[...],
            preferred_element_type=jnp.float32,
        )
        + b1_ref[...]
    )
    h = BM // 2
    acc_a = s1[:h, :]
    acc_b = s1[h:, :]
    a = adj_a_ref[...]
    b = adj_b_ref[...]
    s2 = s2_ref[...]
    for c0 in range(0, N, CK):
        c1 = min(c0 + CK, N)
        s2c = s2[c0:c1, :]
        acc_a = acc_a + jnp.dot(
            a[:, c0:c1], s2c, preferred_element_type=jnp.float32
        )
        acc_b = acc_b + jnp.dot(
            b[:, c0:c1], s2c, preferred_element_type=jnp.float32
        )
    out_ref[:h, :] = acc_a
    out_ref[h:, :] = acc_b


def kernel(input, adj, W1, b1, W2, b2):
    out = pl.pallas_call(
        _fused_body,
        grid=(N // BM,),
        in_specs=[
            pl.BlockSpec((N, D), lambda i: (0, 0)),
            pl.BlockSpec((D, D), lambda i: (0, 0)),
            pl.BlockSpec((1, D), lambda i: (0, 0)),
            pl.BlockSpec((D, D), lambda i: (0, 0)),
            pl.BlockSpec((1, D), lambda i: (0, 0)),
            pl.BlockSpec((BM // 2, N), lambda i: (2 * i, 0)),
            pl.BlockSpec((BM // 2, N), lambda i: (2 * i + 1, 0)),
        ],
        out_specs=pl.BlockSpec((BM, D), lambda i: (i, 0)),
        out_shape=jax.ShapeDtypeStruct((N, D), jnp.float32),
        scratch_shapes=[pltpu.VMEM((N, D), jnp.float32)],
        compiler_params=pltpu.CompilerParams(
            dimension_semantics=("arbitrary",),
        ),
    )(input, W1.T, b1.reshape(1, D), W2.T, b2.reshape(1, D), adj, adj)
    return out
